# baseline (speedup 1.0000x reference)
import jax
import jax.numpy as jnp
from jax import lax
from jax.experimental import pallas as pl
from jax.experimental.pallas import tpu as pltpu
from jax.experimental.pallas import tpu_sc as plsc

V = 10000
E = 320000
NC = 2
NS = 16
CHUNK = 128
GRP = 8
F = 128
NG = -(-2 * E // (NS * GRP * CHUNK))
PAD_ROWS = 128
ACC_ROWS = V + PAD_ROWS
RPT = 624
TAIL0 = NS * RPT


def _tc_matmuls(x, W0t, b0, W1t, b1, relu_in, split_in, w0_scale):
    di, do = W0t.shape
    nsl = do // F
    BV = 2000

    def body(x_ref, w0_ref, b0_ref, w1_ref, b1_ref, y0_ref, y1_ref):
        if split_in:
            xb = jnp.concatenate([x_ref[q] for q in range(di // F)], axis=-1)
        else:
            xb = x_ref[...]
        if relu_in:
            xb = jnp.maximum(xb, 0.0)
        y0 = jnp.dot(xb, w0_ref[...], precision=lax.Precision.HIGHEST,
                     preferred_element_type=jnp.float32) + b0_ref[...]
        if w0_scale != 1.0:
            y0 = y0 * w0_scale
        y1 = jnp.dot(xb, w1_ref[...], precision=lax.Precision.HIGHEST,
                     preferred_element_type=jnp.float32) + b1_ref[...]
        for q in range(nsl):
            y0_ref[q] = y0[:, q * F:(q + 1) * F]
            y1_ref[q] = y1[:, q * F:(q + 1) * F]

    if split_in:
        x_spec = pl.BlockSpec((di // F, BV, F), lambda i: (0, i, 0))
    else:
        x_spec = pl.BlockSpec((BV, di), lambda i: (i, 0))
    return pl.pallas_call(
        body,
        grid=(V // BV,),
        in_specs=[
            x_spec,
            pl.BlockSpec((di, do), lambda i: (0, 0)),
            pl.BlockSpec((1, do), lambda i: (0, 0)),
            pl.BlockSpec((di, do), lambda i: (0, 0)),
            pl.BlockSpec((1, do), lambda i: (0, 0)),
        ],
        out_specs=[
            pl.BlockSpec((nsl, BV, F), lambda i: (0, i, 0)),
            pl.BlockSpec((nsl, BV, F), lambda i: (0, i, 0)),
        ],
        out_shape=[jax.ShapeDtypeStruct((nsl, V, F), jnp.float32)] * 2,
    )(x, W0t, b0[None], W1t, b1[None])


def _tc_add(a, b):
    BV = 2000

    def body(a_ref, b_ref, o_ref):
        o_ref[...] = a_ref[...] + b_ref[...]

    return pl.pallas_call(
        body,
        grid=(V // BV,),
        in_specs=[pl.BlockSpec((BV, F), lambda i: (i, 0))] * 2,
        out_specs=pl.BlockSpec((BV, F), lambda i: (i, 0)),
        out_shape=jax.ShapeDtypeStruct((V, F), jnp.float32),
    )(a, b)


def _sc_aggregate(w0x, w1x, dst_idx, src_idx, esplit):
    nsl = w0x.shape[0]
    n_out = 2 if esplit else nsl
    w0f = w0x.reshape(nsl * V, F)
    w1f = w1x.reshape(nsl * V, F)
    mesh = plsc.VectorSubcoreMesh(core_axis_name="c", subcore_axis_name="s",
                                  num_cores=NC, num_subcores=NS)
    g_per_core = NG // NC

    J = 2 * GRP

    def body(w0x_hbm, w1x_hbm, dst_hbm, src_hbm, out_hbm,
             acc, dst_v, src_v, rows, gsem0, gsem1, ssem0, ssem1):
        c = lax.axis_index("c")
        s = lax.axis_index("s")
        r0 = s * RPT
        if esplit:
            qv = 0
            g_lo = c * g_per_core
            ng2 = g_per_core // 2
        else:
            qv = c * V
            g_lo = 0
            ng2 = NG // 2
        pltpu.sync_copy(w0x_hbm.at[pl.ds(qv + r0, RPT)],
                        acc.at[pl.ds(r0, RPT)])

        @pl.when(s == NS - 1)
        def _():
            pltpu.sync_copy(w0x_hbm.at[pl.ds(qv + TAIL0, V - TAIL0)],
                            acc.at[pl.ds(TAIL0, V - TAIL0)])

        plsc.subcore_barrier()
        gsems = (gsem0, gsem1)
        ssems = (ssem0, ssem1)

        def group2(g2, carry):
            row0 = (s * NG + g_lo + 2 * g2) * GRP
            d1 = pltpu.async_copy(dst_hbm.at[pl.ds(row0, J)], dst_v, ssem0)
            if esplit:
                d2 = pltpu.async_copy(src_hbm.at[pl.ds(row0, J)], src_v,
                                      ssem1)
            else:
                d2 = pltpu.async_copy(
                    src_hbm.at[pl.ds(c * (NS * NG * GRP) + row0, J)], src_v,
                    ssem1)
            d2.wait()
            d1.wait()
            gd = [pltpu.async_copy(w1x_hbm.at[src_v.at[b]], rows.at[b],
                                   gsems[b]) for b in (0, 1)]
            sd = {}
            for jj in range(J):
                b = jj % 2
                gd[b].wait()
                sd[jj] = pltpu.async_copy(rows.at[b], acc.at[dst_v.at[jj]],
                                          ssems[b], add=True)
                if jj + 2 < J:
                    sd[jj].wait()
                    gd[b] = pltpu.async_copy(w1x_hbm.at[src_v.at[jj + 2]],
                                             rows.at[b], gsems[b])
            sd[J - 2].wait()
            sd[J - 1].wait()
            return carry

        lax.fori_loop(0, ng2, group2, 0)
        plsc.subcore_barrier()
        ov = c * V if esplit else qv
        pltpu.sync_copy(acc.at[pl.ds(r0, RPT)],
                        out_hbm.at[pl.ds(ov + r0, RPT)])

        @pl.when(s == NS - 1)
        def _():
            pltpu.sync_copy(acc.at[pl.ds(TAIL0, V - TAIL0)],
                            out_hbm.at[pl.ds(ov + TAIL0, V - TAIL0)])

    out = pl.kernel(
        body,
        out_type=jax.ShapeDtypeStruct((n_out * V, F), jnp.float32),
        mesh=mesh,
        scratch_types=[
            pltpu.VMEM_SHARED((ACC_ROWS, F), jnp.float32),
            pltpu.VMEM((2 * GRP, CHUNK), jnp.int32),
            pltpu.VMEM((2 * GRP, CHUNK), jnp.int32),
            pltpu.VMEM((2, CHUNK, F), jnp.float32),
            pltpu.SemaphoreType.DMA,
            pltpu.SemaphoreType.DMA,
            pltpu.SemaphoreType.DMA,
            pltpu.SemaphoreType.DMA,
        ],
    )(w0f, w1f, dst_idx, src_idx)
    return out.reshape(n_out, V, F)


def _build_incidence(edges):
    s = edges[:, 0]
    d = edges[:, 1]
    dst = jnp.concatenate([s, d])
    src = jnp.concatenate([d, s])
    total = NS * NG * GRP * CHUNK
    pad = total - 2 * E
    dst = jnp.concatenate(
        [dst, V + (jnp.arange(pad, dtype=jnp.int32) % PAD_ROWS)])
    src = jnp.concatenate(
        [src, (jnp.arange(pad, dtype=jnp.int32) * 37) % V])
    dst = dst.reshape(NS * NG * GRP, CHUNK)
    src = src.reshape(NS * NG * GRP, CHUNK)
    src2 = jnp.concatenate([src, src + V])
    return dst, src, src2


def kernel(feats_sampled_verts, edges_packed, W0_0, b0_0, W1_0, b1_0,
           W0_1, b0_1, W1_1, b1_1, W0_2, b0_2, W1_2, b1_2,
           W0_3, b0_3, W1_3, b1_3):
    dst_idx, src_idx, src_idx2 = _build_incidence(edges_packed)
    x = feats_sampled_verts
    layers = [(W0_0, b0_0, W1_0, b1_0), (W0_1, b0_1, W1_1, b1_1),
              (W0_2, b0_2, W1_2, b1_2), (W0_3, b0_3, W1_3, b1_3)]
    for i, (W0, b0, W1, b1) in enumerate(layers):
        last = i == len(layers) - 1
        w0x, w1x = _tc_matmuls(x, W0.T, b0, W1.T, b1,
                               relu_in=(i > 0), split_in=(i > 0),
                               w0_scale=(0.5 if last else 1.0))
        if not last:
            x = _sc_aggregate(w0x, w1x, dst_idx, src_idx2, esplit=False)
        else:
            part = _sc_aggregate(w0x, w1x, dst_idx, src_idx, esplit=True)
            return _tc_add(part[0], part[1])
    return x


# speedup vs baseline: 1.0402x; 1.0402x over previous
import jax
import jax.numpy as jnp
from jax import lax
from jax.experimental import pallas as pl
from jax.experimental.pallas import tpu as pltpu
from jax.experimental.pallas import tpu_sc as plsc

V = 10000
E = 320000
NC = 2
NS = 16
CHUNK = 128
GRP = 16
F = 128
NG = -(-2 * E // (NS * GRP * CHUNK))
PAD_ROWS = 128
ACC_ROWS = V + PAD_ROWS
RPT = 624
TAIL0 = NS * RPT


def _tc_matmuls(x, W0t, b0, W1t, b1, relu_in, split_in, w0_scale):
    di, do = W0t.shape
    nsl = do // F
    BV = 2000

    def body(x_ref, w0_ref, b0_ref, w1_ref, b1_ref, y0_ref, y1_ref):
        if split_in:
            xb = jnp.concatenate([x_ref[q] for q in range(di // F)], axis=-1)
        else:
            xb = x_ref[...]
        if relu_in:
            xb = jnp.maximum(xb, 0.0)
        y0 = jnp.dot(xb, w0_ref[...], precision=lax.Precision.HIGHEST,
                     preferred_element_type=jnp.float32) + b0_ref[...]
        if w0_scale != 1.0:
            y0 = y0 * w0_scale
        y1 = jnp.dot(xb, w1_ref[...], precision=lax.Precision.HIGHEST,
                     preferred_element_type=jnp.float32) + b1_ref[...]
        for q in range(nsl):
            y0_ref[q] = y0[:, q * F:(q + 1) * F]
            y1_ref[q] = y1[:, q * F:(q + 1) * F]

    if split_in:
        x_spec = pl.BlockSpec((di // F, BV, F), lambda i: (0, i, 0))
    else:
        x_spec = pl.BlockSpec((BV, di), lambda i: (i, 0))
    return pl.pallas_call(
        body,
        grid=(V // BV,),
        in_specs=[
            x_spec,
            pl.BlockSpec((di, do), lambda i: (0, 0)),
            pl.BlockSpec((1, do), lambda i: (0, 0)),
            pl.BlockSpec((di, do), lambda i: (0, 0)),
            pl.BlockSpec((1, do), lambda i: (0, 0)),
        ],
        out_specs=[
            pl.BlockSpec((nsl, BV, F), lambda i: (0, i, 0)),
            pl.BlockSpec((nsl, BV, F), lambda i: (0, i, 0)),
        ],
        out_shape=[jax.ShapeDtypeStruct((nsl, V, F), jnp.float32)] * 2,
    )(x, W0t, b0[None], W1t, b1[None])


def _tc_add(a, b):
    BV = 2000

    def body(a_ref, b_ref, o_ref):
        o_ref[...] = a_ref[...] + b_ref[...]

    return pl.pallas_call(
        body,
        grid=(V // BV,),
        in_specs=[pl.BlockSpec((BV, F), lambda i: (i, 0))] * 2,
        out_specs=pl.BlockSpec((BV, F), lambda i: (i, 0)),
        out_shape=jax.ShapeDtypeStruct((V, F), jnp.float32),
    )(a, b)


def _sc_aggregate(w0x, w1x, dst_idx, src_idx, esplit):
    nsl = w0x.shape[0]
    n_out = 2 if esplit else nsl
    w0f = w0x.reshape(nsl * V, F)
    w1f = w1x.reshape(nsl * V, F)
    mesh = plsc.VectorSubcoreMesh(core_axis_name="c", subcore_axis_name="s",
                                  num_cores=NC, num_subcores=NS)
    g_per_core = NG // NC

    J = 2 * GRP

    def body(w0x_hbm, w1x_hbm, dst_hbm, src_hbm, out_hbm,
             acc, dst_v, src_v, rows, gsem0, gsem1, ssem0, ssem1):
        c = lax.axis_index("c")
        s = lax.axis_index("s")
        r0 = s * RPT
        if esplit:
            qv = 0
            g_lo = c * g_per_core
            ng2 = g_per_core // 2
        else:
            qv = c * V
            g_lo = 0
            ng2 = NG // 2
        pltpu.sync_copy(w0x_hbm.at[pl.ds(qv + r0, RPT)],
                        acc.at[pl.ds(r0, RPT)])

        @pl.when(s == NS - 1)
        def _():
            pltpu.sync_copy(w0x_hbm.at[pl.ds(qv + TAIL0, V - TAIL0)],
                            acc.at[pl.ds(TAIL0, V - TAIL0)])

        plsc.subcore_barrier()
        gsems = (gsem0, gsem1)
        ssems = (ssem0, ssem1)

        def group2(g2, carry):
            row0 = (s * NG + g_lo + 2 * g2) * GRP
            d1 = pltpu.async_copy(dst_hbm.at[pl.ds(row0, J)], dst_v, ssem0)
            if esplit:
                d2 = pltpu.async_copy(src_hbm.at[pl.ds(row0, J)], src_v,
                                      ssem1)
            else:
                d2 = pltpu.async_copy(
                    src_hbm.at[pl.ds(c * (NS * NG * GRP) + row0, J)], src_v,
                    ssem1)
            d2.wait()
            d1.wait()
            gd = [pltpu.async_copy(w1x_hbm.at[src_v.at[b]], rows.at[b],
                                   gsems[b]) for b in (0, 1)]
            sd = {}
            for jj in range(J):
                b = jj % 2
                gd[b].wait()
                sd[jj] = pltpu.async_copy(rows.at[b], acc.at[dst_v.at[jj]],
                                          ssems[b], add=True)
                if jj + 2 < J:
                    sd[jj].wait()
                    gd[b] = pltpu.async_copy(w1x_hbm.at[src_v.at[jj + 2]],
                                             rows.at[b], gsems[b])
            sd[J - 2].wait()
            sd[J - 1].wait()
            return carry

        lax.fori_loop(0, ng2, group2, 0)
        plsc.subcore_barrier()
        ov = c * V if esplit else qv
        pltpu.sync_copy(acc.at[pl.ds(r0, RPT)],
                        out_hbm.at[pl.ds(ov + r0, RPT)])

        @pl.when(s == NS - 1)
        def _():
            pltpu.sync_copy(acc.at[pl.ds(TAIL0, V - TAIL0)],
                            out_hbm.at[pl.ds(ov + TAIL0, V - TAIL0)])

    out = pl.kernel(
        body,
        out_type=jax.ShapeDtypeStruct((n_out * V, F), jnp.float32),
        mesh=mesh,
        scratch_types=[
            pltpu.VMEM_SHARED((ACC_ROWS, F), jnp.float32),
            pltpu.VMEM((2 * GRP, CHUNK), jnp.int32),
            pltpu.VMEM((2 * GRP, CHUNK), jnp.int32),
            pltpu.VMEM((2, CHUNK, F), jnp.float32),
            pltpu.SemaphoreType.DMA,
            pltpu.SemaphoreType.DMA,
            pltpu.SemaphoreType.DMA,
            pltpu.SemaphoreType.DMA,
        ],
    )(w0f, w1f, dst_idx, src_idx)
    return out.reshape(n_out, V, F)


def _build_incidence(edges):
    s = edges[:, 0]
    d = edges[:, 1]
    dst = jnp.concatenate([s, d])
    src = jnp.concatenate([d, s])
    total = NS * NG * GRP * CHUNK
    pad = total - 2 * E
    dst = jnp.concatenate(
        [dst, V + (jnp.arange(pad, dtype=jnp.int32) % PAD_ROWS)])
    src = jnp.concatenate(
        [src, (jnp.arange(pad, dtype=jnp.int32) * 37) % V])
    dst = dst.reshape(NS * NG * GRP, CHUNK)
    src = src.reshape(NS * NG * GRP, CHUNK)
    src2 = jnp.concatenate([src, src + V])
    return dst, src, src2


def kernel(feats_sampled_verts, edges_packed, W0_0, b0_0, W1_0, b1_0,
           W0_1, b0_1, W1_1, b1_1, W0_2, b0_2, W1_2, b1_2,
           W0_3, b0_3, W1_3, b1_3):
    dst_idx, src_idx, src_idx2 = _build_incidence(edges_packed)
    x = feats_sampled_verts
    layers = [(W0_0, b0_0, W1_0, b1_0), (W0_1, b0_1, W1_1, b1_1),
              (W0_2, b0_2, W1_2, b1_2), (W0_3, b0_3, W1_3, b1_3)]
    for i, (W0, b0, W1, b1) in enumerate(layers):
        last = i == len(layers) - 1
        w0x, w1x = _tc_matmuls(x, W0.T, b0, W1.T, b1,
                               relu_in=(i > 0), split_in=(i > 0),
                               w0_scale=(0.5 if last else 1.0))
        if not last:
            x = _sc_aggregate(w0x, w1x, dst_idx, src_idx2, esplit=False)
        else:
            part = _sc_aggregate(w0x, w1x, dst_idx, src_idx, esplit=True)
            return _tc_add(part[0], part[1])
    return x
